# baseline (speedup 1.0000x reference)
import jax
import jax.numpy as jnp
from jax.experimental import pallas as pl
from jax.experimental.pallas import tpu as pltpu
from jax.experimental.pallas import tpu_sc as plsc

BATCH = 16384
HIST = 200
EMB = 32
N = BATCH * HIST
WINDOW = 128
K = 8
MB = 4096


def _sc_gather(table_hbm, idx_hbm, out_hbm, sem):
    def body(i_vmem, o_vmem):
        copies = [
            pltpu.async_copy(
                table_hbm.at[i_vmem.at[j]],
                o_vmem.at[pl.ds(j * WINDOW, WINDOW)],
                sem,
            )
            for j in range(K)
        ]
        for c in copies:
            c.wait()

    pltpu.emit_pipeline(
        body,
        grid=(N // (WINDOW * K),),
        in_specs=[pl.BlockSpec((K, WINDOW), index_map=lambda i: (i, 0))],
        out_specs=[pl.BlockSpec((K * WINDOW, EMB), index_map=lambda i: (i, 0))],
        core_axis_name=("c", "s"),
        dimension_semantics=(pltpu.PARALLEL,),
    )(idx_hbm, out_hbm)


def _tc_transpose(x_ref, o_ref):
    o_ref[...] = x_ref[...].T.reshape(4, EMB, MB)


def _impl(prev_action, action_emb_weight):
    idx = (
        jnp.transpose(prev_action.reshape(BATCH, HIST // 4, 4), (1, 0, 2))
        .reshape(N // WINDOW, WINDOW)
        .astype(jnp.int32)
    )
    mesh = plsc.VectorSubcoreMesh(core_axis_name="c", subcore_axis_name="s")
    interm = pl.kernel(
        _sc_gather,
        out_type=jax.ShapeDtypeStruct((N, EMB), jnp.float32),
        mesh=mesh,
        scratch_types=[pltpu.SemaphoreType.DMA],
        compiler_params=pltpu.CompilerParams(use_tc_tiling_on_sc=False),
    )(action_emb_weight, idx)

    x2 = interm.reshape(N * EMB // 128, 128)
    t = pl.pallas_call(
        _tc_transpose,
        out_shape=jax.ShapeDtypeStruct((HIST, EMB, BATCH), jnp.float32),
        grid=(HIST // 4, BATCH // MB),
        in_specs=[
            pl.BlockSpec(
                (MB, 128), lambda j, m: (j * (BATCH // MB) + m, 0)
            )
        ],
        out_specs=pl.BlockSpec((4, EMB, MB), lambda j, m: (j, 0, m)),
        compiler_params=pltpu.CompilerParams(
            dimension_semantics=("parallel", "parallel")
        ),
    )(x2)
    return jnp.transpose(t, (2, 0, 1))


kernel = jax.jit(_impl)


# speedup vs baseline: 1.7448x; 1.7448x over previous
import jax
import jax.numpy as jnp
from jax.experimental import pallas as pl
from jax.experimental.pallas import tpu as pltpu
from jax.experimental.pallas import tpu_sc as plsc

BATCH = 16384
HIST = 200
EMB = 32
N = BATCH * HIST
WINDOW = 128
K = 8
BB = 128


def _sc_gather(table_hbm, idx_hbm, out_hbm, sem):
    def body(i_vmem, o_vmem):
        copies = [
            pltpu.async_copy(
                table_hbm.at[i_vmem.at[j]],
                o_vmem.at[pl.ds(j * WINDOW, WINDOW)],
                sem,
            )
            for j in range(K)
        ]
        for c in copies:
            c.wait()

    pltpu.emit_pipeline(
        body,
        grid=(N // (WINDOW * K),),
        in_specs=[pl.BlockSpec((K, WINDOW), index_map=lambda i: (i, 0))],
        out_specs=[pl.BlockSpec((K * WINDOW, EMB), index_map=lambda i: (i, 0))],
        core_axis_name=("c", "s"),
        dimension_semantics=(pltpu.PARALLEL,),
    )(idx_hbm, out_hbm)


def _tc_transpose(x_ref, o_ref):
    x = x_ref[...].reshape(BB, HIST * EMB // 128, 128)
    t = jnp.transpose(x, (1, 2, 0))
    o_ref[...] = t.reshape(HIST, EMB, BB)


def _impl(prev_action, action_emb_weight):
    idx = prev_action.reshape(N // WINDOW, WINDOW).astype(jnp.int32)
    mesh = plsc.VectorSubcoreMesh(core_axis_name="c", subcore_axis_name="s")
    interm = pl.kernel(
        _sc_gather,
        out_type=jax.ShapeDtypeStruct((N, EMB), jnp.float32),
        mesh=mesh,
        scratch_types=[pltpu.SemaphoreType.DMA],
        compiler_params=pltpu.CompilerParams(use_tc_tiling_on_sc=False),
    )(action_emb_weight, idx)

    x2 = interm.reshape(N * EMB // 128, 128)
    t = pl.pallas_call(
        _tc_transpose,
        out_shape=jax.ShapeDtypeStruct((HIST, EMB, BATCH), jnp.float32),
        grid=(BATCH // BB,),
        in_specs=[
            pl.BlockSpec((BB * HIST * EMB // 128, 128), lambda i: (i, 0))
        ],
        out_specs=pl.BlockSpec((HIST, EMB, BB), lambda i: (0, 0, i)),
    )(x2)
    return jnp.transpose(t, (2, 0, 1))


kernel = jax.jit(_impl)


# speedup vs baseline: 2.5059x; 1.4362x over previous
import jax
import jax.numpy as jnp
from jax.experimental import pallas as pl
from jax.experimental.pallas import tpu as pltpu
from jax.experimental.pallas import tpu_sc as plsc

BATCH = 16384
HIST = 200
EMB = 32
N = BATCH * HIST
WINDOW = 128
K = 8
BB = 128


def _sc_gather(table_hbm, idx_hbm, out_hbm, sem):
    def body(i_vmem, o_vmem):
        copies = [
            pltpu.async_copy(
                table_hbm.at[i_vmem.at[j]],
                o_vmem.at[pl.ds(j * WINDOW, WINDOW)],
                sem,
            )
            for j in range(K)
        ]
        for c in copies:
            c.wait()

    pltpu.emit_pipeline(
        body,
        grid=(N // (WINDOW * K),),
        in_specs=[pl.BlockSpec((K, WINDOW), index_map=lambda i: (i, 0))],
        out_specs=[pl.BlockSpec((K * WINDOW, EMB), index_map=lambda i: (i, 0))],
        core_axis_name=("c", "s"),
        dimension_semantics=(pltpu.PARALLEL,),
    )(idx_hbm, out_hbm)


def _tc_transpose(x_ref, o_ref):
    x = x_ref[...].reshape(BB, HIST * EMB // 128, 128)
    y = jnp.transpose(x, (1, 0, 2))
    t = jnp.transpose(y, (0, 2, 1))
    o_ref[...] = t.reshape(HIST, EMB, BB)


def _impl(prev_action, action_emb_weight):
    idx = prev_action.reshape(N // WINDOW, WINDOW).astype(jnp.int32)
    mesh = plsc.VectorSubcoreMesh(core_axis_name="c", subcore_axis_name="s")
    interm = pl.kernel(
        _sc_gather,
        out_type=jax.ShapeDtypeStruct((N, EMB), jnp.float32),
        mesh=mesh,
        scratch_types=[pltpu.SemaphoreType.DMA],
        compiler_params=pltpu.CompilerParams(use_tc_tiling_on_sc=False),
    )(action_emb_weight, idx)

    x2 = interm.reshape(N * EMB // 128, 128)
    t = pl.pallas_call(
        _tc_transpose,
        out_shape=jax.ShapeDtypeStruct((HIST, EMB, BATCH), jnp.float32),
        grid=(BATCH // BB,),
        in_specs=[
            pl.BlockSpec((BB * HIST * EMB // 128, 128), lambda i: (i, 0))
        ],
        out_specs=pl.BlockSpec((HIST, EMB, BB), lambda i: (0, 0, i)),
    )(x2)
    return jnp.transpose(t, (2, 0, 1))


kernel = jax.jit(_impl)


# speedup vs baseline: 2.6308x; 1.0498x over previous
import jax
import jax.numpy as jnp
from jax.experimental import pallas as pl
from jax.experimental.pallas import tpu as pltpu
from jax.experimental.pallas import tpu_sc as plsc

BATCH = 16384
HIST = 200
EMB = 32
N = BATCH * HIST
WINDOW = 128
K = 8
BB = 256


def _sc_gather(table_hbm, idx_hbm, out_hbm, sem):
    def body(i_vmem, o_vmem):
        copies = [
            pltpu.async_copy(
                table_hbm.at[i_vmem.at[j]],
                o_vmem.at[pl.ds(j * WINDOW, WINDOW)],
                sem,
            )
            for j in range(K)
        ]
        for c in copies:
            c.wait()

    pltpu.emit_pipeline(
        body,
        grid=(N // (WINDOW * K),),
        in_specs=[pl.BlockSpec((K, WINDOW), index_map=lambda i: (i, 0))],
        out_specs=[pl.BlockSpec((K * WINDOW, EMB), index_map=lambda i: (i, 0))],
        core_axis_name=("c", "s"),
        dimension_semantics=(pltpu.PARALLEL,),
    )(idx_hbm, out_hbm)


def _tc_transpose(x_ref, o_ref):
    x = x_ref[...].reshape(BB, HIST * EMB // 128, 128)
    y = jnp.transpose(x, (1, 0, 2))
    t = jnp.transpose(y, (0, 2, 1))
    o_ref[...] = t.reshape(HIST, EMB, BB)


def _impl(prev_action, action_emb_weight):
    idx = prev_action.reshape(N // WINDOW, WINDOW).astype(jnp.int32)
    mesh = plsc.VectorSubcoreMesh(core_axis_name="c", subcore_axis_name="s")
    interm = pl.kernel(
        _sc_gather,
        out_type=jax.ShapeDtypeStruct((N, EMB), jnp.float32),
        mesh=mesh,
        scratch_types=[pltpu.SemaphoreType.DMA],
        compiler_params=pltpu.CompilerParams(use_tc_tiling_on_sc=False),
    )(action_emb_weight, idx)

    x2 = interm.reshape(N * EMB // 128, 128)
    t = pl.pallas_call(
        _tc_transpose,
        out_shape=jax.ShapeDtypeStruct((HIST, EMB, BATCH), jnp.float32),
        grid=(BATCH // BB,),
        in_specs=[
            pl.BlockSpec((BB * HIST * EMB // 128, 128), lambda i: (i, 0))
        ],
        out_specs=pl.BlockSpec((HIST, EMB, BB), lambda i: (0, 0, i)),
    )(x2)
    return jnp.transpose(t, (2, 0, 1))


kernel = jax.jit(_impl)


# speedup vs baseline: 2.6703x; 1.0150x over previous
import jax
import jax.numpy as jnp
from jax.experimental import pallas as pl
from jax.experimental.pallas import tpu as pltpu
from jax.experimental.pallas import tpu_sc as plsc

BATCH = 16384
HIST = 200
EMB = 32
N = BATCH * HIST
WINDOW = 128
K = 8
BB = 512


def _sc_gather(table_hbm, idx_hbm, out_hbm, sem):
    def body(i_vmem, o_vmem):
        copies = [
            pltpu.async_copy(
                table_hbm.at[i_vmem.at[j]],
                o_vmem.at[pl.ds(j * WINDOW, WINDOW)],
                sem,
            )
            for j in range(K)
        ]
        for c in copies:
            c.wait()

    pltpu.emit_pipeline(
        body,
        grid=(N // (WINDOW * K),),
        in_specs=[pl.BlockSpec((K, WINDOW), index_map=lambda i: (i, 0))],
        out_specs=[pl.BlockSpec((K * WINDOW, EMB), index_map=lambda i: (i, 0))],
        core_axis_name=("c", "s"),
        dimension_semantics=(pltpu.PARALLEL,),
    )(idx_hbm, out_hbm)


def _tc_transpose(x_ref, o_ref):
    x = x_ref[...].reshape(BB, HIST * EMB // 128, 128)
    y = jnp.transpose(x, (1, 0, 2))
    t = jnp.transpose(y, (0, 2, 1))
    o_ref[...] = t.reshape(HIST, EMB, BB)


def _impl(prev_action, action_emb_weight):
    idx = prev_action.reshape(N // WINDOW, WINDOW).astype(jnp.int32)
    mesh = plsc.VectorSubcoreMesh(core_axis_name="c", subcore_axis_name="s")
    interm = pl.kernel(
        _sc_gather,
        out_type=jax.ShapeDtypeStruct((N, EMB), jnp.float32),
        mesh=mesh,
        scratch_types=[pltpu.SemaphoreType.DMA],
        compiler_params=pltpu.CompilerParams(use_tc_tiling_on_sc=False),
    )(action_emb_weight, idx)

    x2 = interm.reshape(N * EMB // 128, 128)
    t = pl.pallas_call(
        _tc_transpose,
        out_shape=jax.ShapeDtypeStruct((HIST, EMB, BATCH), jnp.float32),
        grid=(BATCH // BB,),
        in_specs=[
            pl.BlockSpec((BB * HIST * EMB // 128, 128), lambda i: (i, 0))
        ],
        out_specs=pl.BlockSpec((HIST, EMB, BB), lambda i: (0, 0, i)),
    )(x2)
    return jnp.transpose(t, (2, 0, 1))


kernel = jax.jit(_impl)


# speedup vs baseline: 2.6784x; 1.0030x over previous
import jax
import jax.numpy as jnp
from jax.experimental import pallas as pl
from jax.experimental.pallas import tpu as pltpu
from jax.experimental.pallas import tpu_sc as plsc

BATCH = 16384
HIST = 200
EMB = 32
N = BATCH * HIST
WINDOW = 128
K = 8
BB = 512


RB = 8


def _sc_gather(table_hbm, idx_hbm, out_hbm, sem):
    def body(i_vmem, o_vmem):
        copies = [
            pltpu.async_copy(
                table_hbm.at[i_vmem.at[b, pl.ds(o, n)]],
                o_vmem.at[pl.ds(b * HIST + o, n)],
                sem,
            )
            for b in range(RB)
            for (o, n) in ((0, 104), (104, 96))
        ]
        for c in copies:
            c.wait()

    pltpu.emit_pipeline(
        body,
        grid=(BATCH // RB,),
        in_specs=[pl.BlockSpec((RB, HIST), index_map=lambda i: (i, 0))],
        out_specs=[pl.BlockSpec((RB * HIST, EMB), index_map=lambda i: (i, 0))],
        core_axis_name=("c", "s"),
        dimension_semantics=(pltpu.PARALLEL,),
    )(idx_hbm, out_hbm)


def _tc_transpose(x_ref, o_ref):
    x = x_ref[...].reshape(BB, HIST * EMB // 128, 128)
    y = jnp.transpose(x, (1, 0, 2))
    t = jnp.transpose(y, (0, 2, 1))
    o_ref[...] = t.reshape(HIST, EMB, BB)


def _impl(prev_action, action_emb_weight):
    idx = prev_action.astype(jnp.int32)
    mesh = plsc.VectorSubcoreMesh(core_axis_name="c", subcore_axis_name="s")
    interm = pl.kernel(
        _sc_gather,
        out_type=jax.ShapeDtypeStruct((N, EMB), jnp.float32),
        mesh=mesh,
        scratch_types=[pltpu.SemaphoreType.DMA],
        compiler_params=pltpu.CompilerParams(use_tc_tiling_on_sc=False),
    )(action_emb_weight, idx)

    x2 = interm.reshape(N * EMB // 128, 128)
    t = pl.pallas_call(
        _tc_transpose,
        out_shape=jax.ShapeDtypeStruct((HIST, EMB, BATCH), jnp.float32),
        grid=(BATCH // BB,),
        in_specs=[
            pl.BlockSpec((BB * HIST * EMB // 128, 128), lambda i: (i, 0))
        ],
        out_specs=pl.BlockSpec((HIST, EMB, BB), lambda i: (0, 0, i)),
    )(x2)
    return jnp.transpose(t, (2, 0, 1))


kernel = jax.jit(_impl)
